# baseline (speedup 1.0000x reference)
import jax
import jax.numpy as jnp
from jax import lax
from jax.experimental import pallas as pl
from jax.experimental.pallas import tpu as pltpu
from jax.experimental.pallas import tpu_sc as plsc

_B = 16384
_D = 64
_NC = 2
_NS = 16
_NW = _NC * _NS
_BPW = _B // _NW
_CHUNK = 128
_PASS = 256


def _sc_gather(table_padded, idx2d):
    mesh = plsc.VectorSubcoreMesh(core_axis_name="c", subcore_axis_name="s")

    @pl.kernel(
        out_type=jax.ShapeDtypeStruct((_B, 2 * _D), jnp.float32),
        mesh=mesh,
        scratch_types=[
            pltpu.VMEM((_BPW // _CHUNK, _CHUNK), jnp.int32),
            pltpu.VMEM((_PASS, 2 * _D), jnp.float32),
            pltpu.SemaphoreType.DMA,
        ],
    )
    def k(tab_hbm, idx_hbm, out_hbm, idx_v, rows, sem):
        wid = lax.axis_index("s") * _NC + lax.axis_index("c")
        base = wid * _BPW
        nidx = _BPW // _CHUNK
        pltpu.sync_copy(idx_hbm.at[pl.ds(wid * nidx, nidx)], idx_v)
        for p in range(_BPW // _PASS):
            copies = []
            for j in range(_PASS // _CHUNK):
                copies.append(pltpu.async_copy(
                    tab_hbm.at[idx_v.at[p * (_PASS // _CHUNK) + j]],
                    rows.at[pl.ds(j * _CHUNK, _CHUNK)], sem))
            for c in copies:
                c.wait()
            pltpu.sync_copy(rows, out_hbm.at[pl.ds(base + p * _PASS, _PASS)])

    return k(table_padded, idx2d)


def _mlp_body(item_ref, text_ref, w1_ref, b1_ref, w2_ref, b2_ref, out_ref):
    item = item_ref[:, :_D]
    text = text_ref[:, :_D]
    w1 = w1_ref[...]
    cdims = (((1,), (1,)), ((), ()))
    h = lax.dot_general(item, w1[:, :_D], cdims,
                        preferred_element_type=jnp.float32)
    h = h + lax.dot_general(text, w1[:, _D:], cdims,
                            preferred_element_type=jnp.float32)
    h = jnp.maximum(h + b1_ref[...], 0.0)
    w2 = w2_ref[...]
    logits = lax.dot_general(h, w2, cdims, preferred_element_type=jnp.float32)
    b2v = b2_ref[...]
    d = (logits[:, 0:1] - logits[:, 1:2]) + (b2v[0, 0] - b2v[0, 1])
    g0 = 1.0 / (1.0 + jnp.exp(-d))
    out_ref[...] = text + g0 * (item - text)


def _tc_gating(item_emb, text_emb, W1, b1, W2, b2, blk=4096):
    grid = (_B // blk,)
    return pl.pallas_call(
        _mlp_body,
        out_shape=jax.ShapeDtypeStruct((_B, _D), jnp.float32),
        grid=grid,
        in_specs=[
            pl.BlockSpec((blk, 2 * _D), lambda i: (i, 0)),
            pl.BlockSpec((blk, 2 * _D), lambda i: (i, 0)),
            pl.BlockSpec((128, 128), lambda i: (0, 0)),
            pl.BlockSpec((1, 128), lambda i: (0, 0)),
            pl.BlockSpec((2, 128), lambda i: (0, 0)),
            pl.BlockSpec((1, 2), lambda i: (0, 0)),
        ],
        out_specs=pl.BlockSpec((blk, _D), lambda i: (i, 0)),
    )(item_emb, text_emb, W1, b1, W2, b2)


def kernel(item_ids, item_table, text_table, W1, b1, W2, b2):
    idx2d = item_ids.astype(jnp.int32).reshape(_B // _CHUNK, _CHUNK)
    item_pad = jnp.pad(item_table, ((0, 0), (0, _D)))
    text_pad = jnp.pad(text_table, ((0, 0), (0, _D)))
    item_rows = _sc_gather(item_pad, idx2d)
    text_rows = _sc_gather(text_pad, idx2d)
    return _tc_gating(item_rows, text_rows, W1,
                      b1.reshape(1, 128), W2, b2.reshape(1, 2))
